# baseline (speedup 1.0000x reference)
import jax
import jax.numpy as jnp
from jax.experimental import pallas as pl

_NP = 50000
_NA = 50000
_D = 128
_BLK = 400
_WCOLS = 512


def _mm_kernel(x_ref, w_ref, o_ref):
    o_ref[...] = jnp.dot(x_ref[...], w_ref[...],
                         preferred_element_type=jnp.float32)


def _proj(x, w):
    n = x.shape[0]
    return pl.pallas_call(
        _mm_kernel,
        grid=(n // _BLK,),
        in_specs=[pl.BlockSpec((_BLK, _D), lambda i: (i, 0)),
                  pl.BlockSpec((_D, _WCOLS), lambda i: (0, 0))],
        out_specs=pl.BlockSpec((_BLK, _WCOLS), lambda i: (i, 0)),
        out_shape=jax.ShapeDtypeStruct((n, _WCOLS), jnp.float32),
    )(x, w)


def _out_kernel(mm_ref, h1_ref, h2_ref, b_ref, o_ref):
    o_ref[...] = jnp.maximum(
        mm_ref[...] + h1_ref[...] + h2_ref[...] + b_ref[...], 0.0)


def _finish(mm, h1, h2, b):
    n = h1.shape[0]
    return pl.pallas_call(
        _out_kernel,
        grid=(n // _BLK,),
        in_specs=[pl.BlockSpec((_BLK, _D), lambda i: (i, 0)),
                  pl.BlockSpec((_BLK, _D), lambda i: (i, 0)),
                  pl.BlockSpec((_BLK, _D), lambda i: (i, 0)),
                  pl.BlockSpec((1, _D), lambda i: (0, 0))],
        out_specs=pl.BlockSpec((_BLK, _D), lambda i: (i, 0)),
        out_shape=jax.ShapeDtypeStruct((n, _D), jnp.float32),
    )(mm, h1, h2, b.reshape(1, _D))


def _edge(s_src, s_dst, z_src, b_rel, edges, num_dst):
    src = edges[0]
    dst = edges[1]
    e = jax.nn.leaky_relu(s_src[src] + s_dst[dst], negative_slope=0.2)
    m = jax.ops.segment_max(e, dst, num_segments=num_dst)
    m = jnp.where(jnp.isfinite(m), m, 0.0)
    ee = jnp.exp(e - m[dst])
    denom = jax.ops.segment_sum(ee, dst, num_segments=num_dst)
    alpha = ee / denom[dst]
    h = jax.ops.segment_sum(alpha[:, None] * z_src[src], dst,
                            num_segments=num_dst)
    asum = jax.ops.segment_sum(alpha, dst, num_segments=num_dst)
    return h + asum[:, None] * b_rel


def kernel(feat_P, feat_A, edge_p2p, edge_p2a, edge_a2p, edge_a2a,
           W_P, b_P, W_A, b_A, W_p2p, b_p2p, W_p2a, b_p2a,
           W_a2p, b_a2p, W_a2a, b_a2a, a_p2p, a_p2a, a_a2p, a_a2a):
    a1_p2p, a2_p2p = a_p2p[:_D], a_p2p[_D:]
    a1_p2a, a2_p2a = a_p2a[:_D], a_p2a[_D:]
    a1_a2p, a2_a2p = a_a2p[:_D], a_a2p[_D:]
    a1_a2a, a2_a2a = a_a2a[:_D], a_a2a[_D:]

    pad = jnp.zeros((_D, _WCOLS - 388), jnp.float32)
    wP = jnp.concatenate([
        W_P, W_p2p, W_p2a,
        (W_p2p @ a1_p2p)[:, None],
        (W_p2a @ a1_p2a)[:, None],
        (W_P @ a2_p2p)[:, None],
        (W_P @ a2_a2p)[:, None],
        pad], axis=1)
    wA = jnp.concatenate([
        W_A, W_a2p, W_a2a,
        (W_a2p @ a1_a2p)[:, None],
        (W_a2a @ a1_a2a)[:, None],
        (W_A @ a2_p2a)[:, None],
        (W_A @ a2_a2a)[:, None],
        pad], axis=1)

    mmP = _proj(feat_P, wP)
    mmA = _proj(feat_A, wA)

    s_src_p2p = mmP[:, 384] + b_p2p @ a1_p2p
    s_src_p2a = mmP[:, 385] + b_p2a @ a1_p2a
    s_dst_p2p = mmP[:, 386] + b_P @ a2_p2p
    s_dst_a2p = mmP[:, 387] + b_P @ a2_a2p
    s_src_a2p = mmA[:, 384] + b_a2p @ a1_a2p
    s_src_a2a = mmA[:, 385] + b_a2a @ a1_a2a
    s_dst_p2a = mmA[:, 386] + b_A @ a2_p2a
    s_dst_a2a = mmA[:, 387] + b_A @ a2_a2a

    z_p2p = mmP[:, 128:256]
    z_p2a = mmP[:, 256:384]
    z_a2p = mmA[:, 128:256]
    z_a2a = mmA[:, 256:384]

    h_p2p = _edge(s_src_p2p, s_dst_p2p, z_p2p, b_p2p, edge_p2p, _NP)
    h_p2a = _edge(s_src_p2a, s_dst_p2a, z_p2a, b_p2a, edge_p2a, _NA)
    h_a2p = _edge(s_src_a2p, s_dst_a2p, z_a2p, b_a2p, edge_a2p, _NP)
    h_a2a = _edge(s_src_a2a, s_dst_a2a, z_a2a, b_a2a, edge_a2a, _NA)

    out_P = _finish(mmP, h_p2p, h_a2p, b_P)
    out_A = _finish(mmA, h_p2a, h_a2a, b_A)
    return (out_P, out_A)


# speedup vs baseline: 1.0033x; 1.0033x over previous
import jax
import jax.numpy as jnp
from jax.experimental import pallas as pl

_NP = 50000
_NA = 50000
_D = 128
_BLK = 400


def _mm_kernel(x_ref, w_ref, ws_ref, wh_ref, z1_ref, z2_ref, sc_ref):
    big = jnp.dot(x_ref[...], w_ref[...], preferred_element_type=jnp.float32)
    wh_ref[...] = big[:, :_D]
    z1_ref[...] = big[:, _D:2 * _D]
    z2_ref[...] = big[:, 2 * _D:3 * _D]
    sc_ref[...] = jnp.dot(x_ref[...], ws_ref[...],
                          preferred_element_type=jnp.float32)


def _proj(x, w, ws):
    n = x.shape[0]
    blk = lambda c: pl.BlockSpec((_BLK, c), lambda i: (i, 0))
    return pl.pallas_call(
        _mm_kernel,
        grid=(n // _BLK,),
        in_specs=[blk(_D),
                  pl.BlockSpec((_D, 3 * _D), lambda i: (0, 0)),
                  pl.BlockSpec((_D, 8), lambda i: (0, 0))],
        out_specs=[blk(_D), blk(_D), blk(_D), blk(8)],
        out_shape=[jax.ShapeDtypeStruct((n, _D), jnp.float32),
                   jax.ShapeDtypeStruct((n, _D), jnp.float32),
                   jax.ShapeDtypeStruct((n, _D), jnp.float32),
                   jax.ShapeDtypeStruct((n, 8), jnp.float32)],
    )(x, w, ws)


def _out_kernel(mm_ref, h1_ref, h2_ref, b_ref, o_ref):
    o_ref[...] = jnp.maximum(
        mm_ref[...] + h1_ref[...] + h2_ref[...] + b_ref[...], 0.0)


def _finish(mm, h1, h2, b):
    n = h1.shape[0]
    return pl.pallas_call(
        _out_kernel,
        grid=(n // _BLK,),
        in_specs=[pl.BlockSpec((_BLK, _D), lambda i: (i, 0)),
                  pl.BlockSpec((_BLK, _D), lambda i: (i, 0)),
                  pl.BlockSpec((_BLK, _D), lambda i: (i, 0)),
                  pl.BlockSpec((1, _D), lambda i: (0, 0))],
        out_specs=pl.BlockSpec((_BLK, _D), lambda i: (i, 0)),
        out_shape=jax.ShapeDtypeStruct((n, _D), jnp.float32),
    )(mm, h1, h2, b.reshape(1, _D))


def _edge(s_src, s_dst, z_src, b_rel, edges, num_dst):
    src = edges[0]
    dst = edges[1]
    e = jax.nn.leaky_relu(s_src[src] + s_dst[dst], negative_slope=0.2)
    m = jax.ops.segment_max(e, dst, num_segments=num_dst)
    m = jnp.where(jnp.isfinite(m), m, 0.0)
    ee = jnp.exp(e - m[dst])
    denom = jax.ops.segment_sum(ee, dst, num_segments=num_dst)
    alpha = ee / denom[dst]
    h = jax.ops.segment_sum(alpha[:, None] * z_src[src], dst,
                            num_segments=num_dst)
    asum = jax.ops.segment_sum(alpha, dst, num_segments=num_dst)
    return h + asum[:, None] * b_rel


def kernel(feat_P, feat_A, edge_p2p, edge_p2a, edge_a2p, edge_a2a,
           W_P, b_P, W_A, b_A, W_p2p, b_p2p, W_p2a, b_p2a,
           W_a2p, b_a2p, W_a2a, b_a2a, a_p2p, a_p2a, a_a2p, a_a2a):
    a1_p2p, a2_p2p = a_p2p[:_D], a_p2p[_D:]
    a1_p2a, a2_p2a = a_p2a[:_D], a_p2a[_D:]
    a1_a2p, a2_a2p = a_a2p[:_D], a_a2p[_D:]
    a1_a2a, a2_a2a = a_a2a[:_D], a_a2a[_D:]

    pad = jnp.zeros((_D, 4), jnp.float32)
    wP = jnp.concatenate([W_P, W_p2p, W_p2a], axis=1)
    wA = jnp.concatenate([W_A, W_a2p, W_a2a], axis=1)
    wsP = jnp.concatenate([
        (W_p2p @ a1_p2p)[:, None],
        (W_p2a @ a1_p2a)[:, None],
        (W_P @ a2_p2p)[:, None],
        (W_P @ a2_a2p)[:, None],
        pad], axis=1)
    wsA = jnp.concatenate([
        (W_a2p @ a1_a2p)[:, None],
        (W_a2a @ a1_a2a)[:, None],
        (W_A @ a2_p2a)[:, None],
        (W_A @ a2_a2a)[:, None],
        pad], axis=1)

    whP, z_p2p, z_p2a, scP = _proj(feat_P, wP, wsP)
    whA, z_a2p, z_a2a, scA = _proj(feat_A, wA, wsA)

    s_src_p2p = scP[:, 0] + b_p2p @ a1_p2p
    s_src_p2a = scP[:, 1] + b_p2a @ a1_p2a
    s_dst_p2p = scP[:, 2] + b_P @ a2_p2p
    s_dst_a2p = scP[:, 3] + b_P @ a2_a2p
    s_src_a2p = scA[:, 0] + b_a2p @ a1_a2p
    s_src_a2a = scA[:, 1] + b_a2a @ a1_a2a
    s_dst_p2a = scA[:, 2] + b_A @ a2_p2a
    s_dst_a2a = scA[:, 3] + b_A @ a2_a2a

    h_p2p = _edge(s_src_p2p, s_dst_p2p, z_p2p, b_p2p, edge_p2p, _NP)
    h_p2a = _edge(s_src_p2a, s_dst_p2a, z_p2a, b_p2a, edge_p2a, _NA)
    h_a2p = _edge(s_src_a2p, s_dst_a2p, z_a2p, b_a2p, edge_a2p, _NP)
    h_a2a = _edge(s_src_a2a, s_dst_a2a, z_a2a, b_a2a, edge_a2a, _NA)

    out_P = _finish(whP, h_p2p, h_a2p, b_P)
    out_A = _finish(whA, h_p2a, h_a2a, b_A)
    return (out_P, out_A)


# speedup vs baseline: 1.3734x; 1.3689x over previous
import jax
import jax.numpy as jnp
from jax.experimental import pallas as pl

_NP = 50000
_NA = 50000
_D = 128
_BLK = 400


def _mm_kernel(x_ref, w_ref, ws_ref, wh_ref, z1_ref, z2_ref, sc_ref):
    big = jnp.dot(x_ref[...], w_ref[...], preferred_element_type=jnp.float32)
    wh_ref[...] = big[:, :_D]
    z1_ref[...] = big[:, _D:2 * _D]
    z2_ref[...] = big[:, 2 * _D:3 * _D]
    sc_ref[...] = jnp.dot(x_ref[...], ws_ref[...],
                          preferred_element_type=jnp.float32)


def _proj(x, w, ws):
    n = x.shape[0]
    blk = lambda c: pl.BlockSpec((_BLK, c), lambda i: (i, 0))
    return pl.pallas_call(
        _mm_kernel,
        grid=(n // _BLK,),
        in_specs=[blk(_D),
                  pl.BlockSpec((_D, 3 * _D), lambda i: (0, 0)),
                  pl.BlockSpec((_D, 8), lambda i: (0, 0))],
        out_specs=[blk(_D), blk(_D), blk(_D), blk(8)],
        out_shape=[jax.ShapeDtypeStruct((n, _D), jnp.float32),
                   jax.ShapeDtypeStruct((n, _D), jnp.float32),
                   jax.ShapeDtypeStruct((n, _D), jnp.float32),
                   jax.ShapeDtypeStruct((n, 8), jnp.float32)],
    )(x, w, ws)


def _out_kernel(mm_ref, h1_ref, h2_ref, b_ref, o_ref):
    o_ref[...] = jnp.maximum(
        mm_ref[...] + h1_ref[...] + h2_ref[...] + b_ref[...], 0.0)


def _finish(mm, h1, h2, b):
    n = h1.shape[0]
    return pl.pallas_call(
        _out_kernel,
        grid=(n // _BLK,),
        in_specs=[pl.BlockSpec((_BLK, _D), lambda i: (i, 0)),
                  pl.BlockSpec((_BLK, _D), lambda i: (i, 0)),
                  pl.BlockSpec((_BLK, _D), lambda i: (i, 0)),
                  pl.BlockSpec((1, _D), lambda i: (0, 0))],
        out_specs=pl.BlockSpec((_BLK, _D), lambda i: (i, 0)),
        out_shape=jax.ShapeDtypeStruct((n, _D), jnp.float32),
    )(mm, h1, h2, b.reshape(1, _D))


def _edge(sc_src, col_s, sc_dst, col_d, const, z_src, b_rel, edges, num_dst):
    src = edges[0]
    dst = edges[1]
    gs = sc_src[src]
    gd = sc_dst[dst]
    e = jax.nn.leaky_relu(gs[:, col_s] + gd[:, col_d] + const,
                          negative_slope=0.2)
    m = jax.ops.segment_max(e, dst, num_segments=num_dst)
    m = jnp.where(jnp.isfinite(m), m, 0.0)
    ee = jnp.exp(e - m[dst])
    denom = jax.ops.segment_sum(ee, dst, num_segments=num_dst)
    alpha = ee / denom[dst]
    h = jax.ops.segment_sum(alpha[:, None] * z_src[src], dst,
                            num_segments=num_dst)
    asum = jnp.where(denom > 0, 1.0, 0.0)
    return h + asum[:, None] * b_rel


def kernel(feat_P, feat_A, edge_p2p, edge_p2a, edge_a2p, edge_a2a,
           W_P, b_P, W_A, b_A, W_p2p, b_p2p, W_p2a, b_p2a,
           W_a2p, b_a2p, W_a2a, b_a2a, a_p2p, a_p2a, a_a2p, a_a2a):
    a1_p2p, a2_p2p = a_p2p[:_D], a_p2p[_D:]
    a1_p2a, a2_p2a = a_p2a[:_D], a_p2a[_D:]
    a1_a2p, a2_a2p = a_a2p[:_D], a_a2p[_D:]
    a1_a2a, a2_a2a = a_a2a[:_D], a_a2a[_D:]

    pad = jnp.zeros((_D, 4), jnp.float32)
    wP = jnp.concatenate([W_P, W_p2p, W_p2a], axis=1)
    wA = jnp.concatenate([W_A, W_a2p, W_a2a], axis=1)
    wsP = jnp.concatenate([
        (W_p2p @ a1_p2p)[:, None],
        (W_p2a @ a1_p2a)[:, None],
        (W_P @ a2_p2p)[:, None],
        (W_P @ a2_a2p)[:, None],
        pad], axis=1)
    wsA = jnp.concatenate([
        (W_a2p @ a1_a2p)[:, None],
        (W_a2a @ a1_a2a)[:, None],
        (W_A @ a2_p2a)[:, None],
        (W_A @ a2_a2a)[:, None],
        pad], axis=1)

    whP, z_p2p, z_p2a, scP = _proj(feat_P, wP, wsP)
    whA, z_a2p, z_a2a, scA = _proj(feat_A, wA, wsA)

    c_p2p = b_p2p @ a1_p2p + b_P @ a2_p2p
    c_p2a = b_p2a @ a1_p2a + b_A @ a2_p2a
    c_a2p = b_a2p @ a1_a2p + b_P @ a2_a2p
    c_a2a = b_a2a @ a1_a2a + b_A @ a2_a2a

    h_p2p = _edge(scP, 0, scP, 2, c_p2p, z_p2p, b_p2p, edge_p2p, _NP)
    h_p2a = _edge(scP, 1, scA, 2, c_p2a, z_p2a, b_p2a, edge_p2a, _NA)
    h_a2p = _edge(scA, 0, scP, 3, c_a2p, z_a2p, b_a2p, edge_a2p, _NP)
    h_a2a = _edge(scA, 1, scA, 3, c_a2a, z_a2a, b_a2a, edge_a2a, _NA)

    out_P = _finish(whP, h_p2p, h_a2p, b_P)
    out_A = _finish(whA, h_p2a, h_a2a, b_A)
    return (out_P, out_A)
